# Initial kernel scaffold; baseline (speedup 1.0000x reference)
import jax
import jax.numpy as jnp
from jax.experimental import pallas as pl


def kernel(x, edge_index, W_emb, b_emb, Wl1, Wr1, b1, g1, be1, Wl2, Wr2, b2, g2, be2, Wl3, Wr3, b3, g3, be3, Wc, bc):
    raise NotImplementedError("write your pallas kernel here")


# speedup vs baseline: 6.9376x; 6.9376x over previous
import functools

import jax
import jax.numpy as jnp
from jax import lax
from jax.experimental import pallas as pl
from jax.experimental.pallas import tpu as pltpu
from jax.experimental.pallas import tpu_sc as plsc

_NC, _NS = 2, 16
_NW = _NC * _NS
_CH = 80


_MESH = None


def _sc_mesh():
  global _MESH
  if _MESH is None:
    _MESH = plsc.VectorSubcoreMesh(
        core_axis_name="c", subcore_axis_name="s",
        num_cores=_NC, num_subcores=_NS)
  return _MESH


def _make_sc_agg(np_, h, nch):
  rps = np_ // _NS
  nstripe = rps // _CH

  scratch = [
      pltpu.VMEM((nch, _CH), jnp.int32),
      pltpu.VMEM((nch, _CH), jnp.int32),
      pltpu.VMEM((_CH, h), jnp.float32),
      pltpu.VMEM_SHARED((np_, h), jnp.float32),
      pltpu.SemaphoreType.DMA,
  ]

  def body(h_hbm, src_hbm, dst_hbm, out_hbm, src_v, dst_v, rows_v, agg_sh,
           sem):
    c = lax.axis_index("c")
    s = lax.axis_index("s")
    wid = s * _NC + c

    pltpu.sync_copy(src_hbm.at[wid], src_v)
    pltpu.sync_copy(dst_hbm.at[wid], dst_v)

    def zrow(i, carry):
      for k in range(h // 16):
        rows_v[i, pl.ds(k * 16, 16)] = jnp.zeros((16,), jnp.float32)
      return carry

    lax.fori_loop(0, _CH, zrow, 0)
    for k in range(nstripe):
      pltpu.sync_copy(rows_v, agg_sh.at[pl.ds(s * rps + k * _CH, _CH)])
    plsc.subcore_barrier()

    def step(j, carry):
      pltpu.async_copy(h_hbm.at[src_v.at[j]], rows_v, sem).wait()
      pltpu.sync_copy(rows_v, agg_sh.at[dst_v.at[j]], add=True)
      return carry

    lax.fori_loop(0, nch, step, 0)
    plsc.subcore_barrier()
    for k in range(nstripe):
      rs = s * rps + k * _CH
      pltpu.sync_copy(agg_sh.at[pl.ds(rs, _CH)], rows_v)
      pltpu.sync_copy(rows_v, out_hbm.at[c].at[pl.ds(rs, _CH)])

  return pl.kernel(
      body,
      out_type=jax.ShapeDtypeStruct((_NC, np_, h), jnp.float32),
      mesh=_sc_mesh(), scratch_types=scratch)


def _make_sc_deg(np_, nch):
  rps = np_ // _NS
  nstripe = rps // _CH

  scratch = [
      pltpu.VMEM((nch, _CH), jnp.int32),
      pltpu.VMEM((_CH, 128), jnp.float32),
      pltpu.VMEM_SHARED((np_, 128), jnp.float32),
  ]

  def body(dst_hbm, cnt_hbm, dst_v, ones_v, cnt_sh):
    c = lax.axis_index("c")
    s = lax.axis_index("s")
    wid = s * _NC + c

    pltpu.sync_copy(dst_hbm.at[wid], dst_v)

    def fill(val):
      def fb(i, carry):
        for k in range(8):
          ones_v[i, pl.ds(k * 16, 16)] = jnp.full((16,), val, jnp.float32)
        return carry
      lax.fori_loop(0, _CH, fb, 0)

    fill(0.0)
    for k in range(nstripe):
      pltpu.sync_copy(ones_v, cnt_sh.at[pl.ds(s * rps + k * _CH, _CH)])
    fill(1.0)
    plsc.subcore_barrier()

    def step(j, carry):
      pltpu.sync_copy(ones_v, cnt_sh.at[dst_v.at[j]], add=True)
      return carry

    lax.fori_loop(0, nch, step, 0)
    plsc.subcore_barrier()
    for k in range(nstripe):
      rs = s * rps + k * _CH
      pltpu.sync_copy(cnt_sh.at[pl.ds(rs, _CH)], ones_v)
      pltpu.sync_copy(ones_v, cnt_hbm.at[c].at[pl.ds(rs, _CH)])

  return pl.kernel(
      body,
      out_type=jax.ShapeDtypeStruct((_NC, np_, 128), jnp.float32),
      mesh=_sc_mesh(), scratch_types=scratch)


def _embed_body(x_ref, w_ref, b_ref, o_ref):
  o_ref[...] = (jnp.dot(x_ref[...], w_ref[...],
                        preferred_element_type=jnp.float32) + b_ref[...])


def _combine_norm(p, cnt, h, wl, wr, b, g, be):
  n = h.shape[0]
  agg = p[0][:n] + p[1][:n]
  deg = cnt[0][:n, 0:1] + cnt[1][:n, 0:1]
  inv = 1.0 / jnp.maximum(deg, 1.0)
  y = (jnp.dot(agg * inv, wl, preferred_element_type=jnp.float32)
       + jnp.dot(h, wr, preferred_element_type=jnp.float32) + b)
  mu = jnp.mean(y, axis=0, keepdims=True)
  var = jnp.mean((y - mu) * (y - mu), axis=0, keepdims=True)
  yn = (y - mu) * lax.rsqrt(var + 1e-5) * g + be
  return jnp.maximum(yn, 0.0)


def _layer_body(p_ref, cnt_ref, h_ref, wl_ref, wr_ref, b_ref, g_ref, be_ref,
                o_ref):
  o_ref[...] = _combine_norm(p_ref[...], cnt_ref[...], h_ref[...],
                             wl_ref[...], wr_ref[...], b_ref[...],
                             g_ref[...], be_ref[...])


def _final_body(p_ref, cnt_ref, h_ref, wl_ref, wr_ref, b_ref, g_ref, be_ref,
                wc_ref, bc_ref, o_ref):
  hr = _combine_norm(p_ref[...], cnt_ref[...], h_ref[...],
                     wl_ref[...], wr_ref[...], b_ref[...],
                     g_ref[...], be_ref[...])
  z = jnp.dot(hr, wc_ref[...], preferred_element_type=jnp.float32) + bc_ref[...]
  m = jnp.max(z, axis=-1, keepdims=True)
  ez = jnp.exp(z - m)
  o_ref[...] = ez / jnp.sum(ez, axis=-1, keepdims=True)


def kernel(x, edge_index, W_emb, b_emb, Wl1, Wr1, b1, g1, be1,
           Wl2, Wr2, b2, g2, be2, Wl3, Wr3, b3, g3, be3, Wc, bc):
  n, d = x.shape
  h = W_emb.shape[1]
  o = Wc.shape[1]
  e = edge_index.shape[1]
  epw = e // _NW
  nch = epw // _CH

  blk = _NS * _CH
  np_ = ((n + blk - 1) // blk) * blk

  src3 = edge_index[0].reshape(_NW, nch, _CH)
  dst3 = edge_index[1].reshape(_NW, nch, _CH)

  sc_agg = _make_sc_agg(np_, h, nch)
  sc_deg = _make_sc_deg(np_, nch)

  f32 = jnp.float32
  embed = pl.pallas_call(
      _embed_body, out_shape=jax.ShapeDtypeStruct((n, h), f32))
  layer = pl.pallas_call(
      _layer_body, out_shape=jax.ShapeDtypeStruct((n, h), f32))
  final = pl.pallas_call(
      _final_body, out_shape=jax.ShapeDtypeStruct((n, o), f32))

  cnt = sc_deg(dst3)
  h0 = embed(x, W_emb, b_emb.reshape(1, h))
  p1 = sc_agg(h0, src3, dst3)
  h1 = layer(p1, cnt, h0, Wl1, Wr1, b1.reshape(1, h), g1.reshape(1, h),
             be1.reshape(1, h))
  p2 = sc_agg(h1, src3, dst3)
  h2 = layer(p2, cnt, h1, Wl2, Wr2, b2.reshape(1, h), g2.reshape(1, h),
             be2.reshape(1, h))
  p3 = sc_agg(h2, src3, dst3)
  return final(p3, cnt, h2, Wl3, Wr3, b3.reshape(1, h), g3.reshape(1, h),
               be3.reshape(1, h), Wc, bc.reshape(1, o))


# speedup vs baseline: 10.4646x; 1.5084x over previous
import functools

import jax
import jax.numpy as jnp
from jax import lax
from jax.experimental import pallas as pl
from jax.experimental.pallas import tpu as pltpu
from jax.experimental.pallas import tpu_sc as plsc

_NC, _NS = 2, 16
_NW = _NC * _NS
_CH = 80


_MESH = None


def _sc_mesh():
  global _MESH
  if _MESH is None:
    _MESH = plsc.VectorSubcoreMesh(
        core_axis_name="c", subcore_axis_name="s",
        num_cores=_NC, num_subcores=_NS)
  return _MESH


def _make_sc_agg(np_, h, nch):
  rps = np_ // _NS
  nstripe = rps // _CH

  scratch = [
      pltpu.VMEM((nch, _CH), jnp.int32),
      pltpu.VMEM((2, _CH), jnp.int32),
      pltpu.VMEM((_CH,), jnp.int32),
      pltpu.VMEM((_CH, h), jnp.float32),
      pltpu.VMEM((_CH, h), jnp.float32),
      pltpu.VMEM_SHARED((np_, h), jnp.float32),
      pltpu.SemaphoreType.DMA,
      pltpu.SemaphoreType.DMA,
  ]

  def body(h_hbm, pk_hbm, out_hbm, pk_v, srci_v, dsti_v, rows_a, rows_b,
           agg_sh, sem_a, sem_b):
    c = lax.axis_index("c")
    s = lax.axis_index("s")
    wid = s * _NC + c
    bufs = (rows_a, rows_b)
    sems = (sem_a, sem_b)

    pltpu.sync_copy(pk_hbm.at[wid], pk_v)

    def unpack_src(j, slot):
      for k in range(_CH // 16):
        v = pk_v[j, pl.ds(k * 16, 16)]
        srci_v[slot, pl.ds(k * 16, 16)] = v & 0xFFFF

    def unpack_dst(j):
      for k in range(_CH // 16):
        v = pk_v[j, pl.ds(k * 16, 16)]
        dsti_v[pl.ds(k * 16, 16)] = lax.shift_right_logical(v, 16)

    def zrow(i, carry):
      for k in range(h // 16):
        rows_a[i, pl.ds(k * 16, 16)] = jnp.zeros((16,), jnp.float32)
      return carry

    lax.fori_loop(0, _CH, zrow, 0)
    for k in range(nstripe):
      pltpu.sync_copy(rows_a, agg_sh.at[pl.ds(s * rps + k * _CH, _CH)])
    plsc.subcore_barrier()

    unpack_src(0, 0)
    pltpu.async_copy(h_hbm.at[srci_v.at[0]], rows_a, sem_a)
    unpack_src(1, 1)
    pltpu.async_copy(h_hbm.at[srci_v.at[1]], rows_b, sem_b)

    def step_pair(i2, carry):
      for b in range(2):
        j = i2 * 2 + b

        @pl.when(j < nch)
        def _():
          pltpu.make_async_copy(
              h_hbm.at[srci_v.at[b]], bufs[b], sems[b]).wait()
          unpack_dst(j)
          pltpu.sync_copy(bufs[b], agg_sh.at[dsti_v], add=True)

          @pl.when(j + 2 < nch)
          def _():
            unpack_src(j + 2, b)
            pltpu.async_copy(h_hbm.at[srci_v.at[b]], bufs[b], sems[b])
      return carry

    lax.fori_loop(0, (nch + 1) // 2, step_pair, 0)
    plsc.subcore_barrier()
    for k in range(nstripe):
      rs = s * rps + k * _CH
      pltpu.sync_copy(agg_sh.at[pl.ds(rs, _CH)], rows_a)
      pltpu.sync_copy(rows_a, out_hbm.at[c].at[pl.ds(rs, _CH)])

  return pl.kernel(
      body,
      out_type=jax.ShapeDtypeStruct((_NC, np_, h), jnp.float32),
      mesh=_sc_mesh(), scratch_types=scratch)


def _make_sc_deg(np_, nch):
  rps = np_ // _NS
  nstripe = rps // _CH

  scratch = [
      pltpu.VMEM((nch, _CH), jnp.int32),
      pltpu.VMEM((_CH, 128), jnp.float32),
      pltpu.VMEM_SHARED((np_, 128), jnp.float32),
  ]

  def body(dst_hbm, cnt_hbm, dst_v, ones_v, cnt_sh):
    c = lax.axis_index("c")
    s = lax.axis_index("s")
    wid = s * _NC + c

    pltpu.sync_copy(dst_hbm.at[wid], dst_v)

    def fill(val):
      def fb(i, carry):
        for k in range(8):
          ones_v[i, pl.ds(k * 16, 16)] = jnp.full((16,), val, jnp.float32)
        return carry
      lax.fori_loop(0, _CH, fb, 0)

    fill(0.0)
    for k in range(nstripe):
      pltpu.sync_copy(ones_v, cnt_sh.at[pl.ds(s * rps + k * _CH, _CH)])
    fill(1.0)
    plsc.subcore_barrier()

    def step(j, carry):
      pltpu.sync_copy(ones_v, cnt_sh.at[dst_v.at[j]], add=True)
      return carry

    lax.fori_loop(0, nch, step, 0)
    plsc.subcore_barrier()
    for k in range(nstripe):
      rs = s * rps + k * _CH
      pltpu.sync_copy(cnt_sh.at[pl.ds(rs, _CH)], ones_v)
      pltpu.sync_copy(ones_v, cnt_hbm.at[c].at[pl.ds(rs, _CH)])

  return pl.kernel(
      body,
      out_type=jax.ShapeDtypeStruct((_NC, np_, 128), jnp.float32),
      mesh=_sc_mesh(), scratch_types=scratch)


def _embed_body(x_ref, w_ref, b_ref, o_ref):
  o_ref[...] = (jnp.dot(x_ref[...], w_ref[...],
                        preferred_element_type=jnp.float32) + b_ref[...])


def _combine_norm(p, cnt, h, wl, wr, b, g, be):
  n = h.shape[0]
  agg = p[0][:n] + p[1][:n]
  deg = cnt[0][:n, 0:1] + cnt[1][:n, 0:1]
  inv = 1.0 / jnp.maximum(deg, 1.0)
  y = (jnp.dot(agg * inv, wl, preferred_element_type=jnp.float32)
       + jnp.dot(h, wr, preferred_element_type=jnp.float32) + b)
  mu = jnp.mean(y, axis=0, keepdims=True)
  var = jnp.mean((y - mu) * (y - mu), axis=0, keepdims=True)
  yn = (y - mu) * lax.rsqrt(var + 1e-5) * g + be
  return jnp.maximum(yn, 0.0)


def _layer_body(p_ref, cnt_ref, h_ref, wl_ref, wr_ref, b_ref, g_ref, be_ref,
                o_ref):
  o_ref[...] = _combine_norm(p_ref[...], cnt_ref[...], h_ref[...],
                             wl_ref[...], wr_ref[...], b_ref[...],
                             g_ref[...], be_ref[...])


def _final_body(p_ref, cnt_ref, h_ref, wl_ref, wr_ref, b_ref, g_ref, be_ref,
                wc_ref, bc_ref, o_ref):
  hr = _combine_norm(p_ref[...], cnt_ref[...], h_ref[...],
                     wl_ref[...], wr_ref[...], b_ref[...],
                     g_ref[...], be_ref[...])
  z = jnp.dot(hr, wc_ref[...], preferred_element_type=jnp.float32) + bc_ref[...]
  m = jnp.max(z, axis=-1, keepdims=True)
  ez = jnp.exp(z - m)
  o_ref[...] = ez / jnp.sum(ez, axis=-1, keepdims=True)


def kernel(x, edge_index, W_emb, b_emb, Wl1, Wr1, b1, g1, be1,
           Wl2, Wr2, b2, g2, be2, Wl3, Wr3, b3, g3, be3, Wc, bc):
  n, d = x.shape
  h = W_emb.shape[1]
  o = Wc.shape[1]
  e = edge_index.shape[1]
  epw = e // _NW
  nch = epw // _CH

  blk = _NS * _CH
  np_ = ((n + blk - 1) // blk) * blk

  dst3 = edge_index[1].reshape(_NW, nch, _CH)
  packed3 = jnp.bitwise_or(
      jnp.left_shift(dst3, 16), edge_index[0].reshape(_NW, nch, _CH))

  sc_agg = _make_sc_agg(np_, h, nch)
  sc_deg = _make_sc_deg(np_, nch)

  f32 = jnp.float32
  embed = pl.pallas_call(
      _embed_body, out_shape=jax.ShapeDtypeStruct((n, h), f32))
  layer = pl.pallas_call(
      _layer_body, out_shape=jax.ShapeDtypeStruct((n, h), f32))
  final = pl.pallas_call(
      _final_body, out_shape=jax.ShapeDtypeStruct((n, o), f32))

  cnt = sc_deg(dst3)
  h0 = embed(x, W_emb, b_emb.reshape(1, h))
  p1 = sc_agg(h0, packed3)
  h1 = layer(p1, cnt, h0, Wl1, Wr1, b1.reshape(1, h), g1.reshape(1, h),
             be1.reshape(1, h))
  p2 = sc_agg(h1, packed3)
  h2 = layer(p2, cnt, h1, Wl2, Wr2, b2.reshape(1, h), g2.reshape(1, h),
             be2.reshape(1, h))
  p3 = sc_agg(h2, packed3)
  return final(p3, cnt, h2, Wl3, Wr3, b3.reshape(1, h), g3.reshape(1, h),
               be3.reshape(1, h), Wc, bc.reshape(1, o))


# speedup vs baseline: 11.6709x; 1.1153x over previous
import functools

import jax
import jax.numpy as jnp
from jax import lax
from jax.experimental import pallas as pl
from jax.experimental.pallas import tpu as pltpu
from jax.experimental.pallas import tpu_sc as plsc

_NC, _NS = 2, 16
_NW = _NC * _NS
_CH = 80


_MESH = None


def _sc_mesh():
  global _MESH
  if _MESH is None:
    _MESH = plsc.VectorSubcoreMesh(
        core_axis_name="c", subcore_axis_name="s",
        num_cores=_NC, num_subcores=_NS)
  return _MESH


def _make_sc_agg(np_, h, nch):
  rps = np_ // _NS
  nstripe = rps // _CH

  nb = 3

  scratch = [
      pltpu.VMEM((nch, _CH), jnp.int32),
      pltpu.VMEM((nb, _CH), jnp.int32),
      pltpu.VMEM((nb, _CH), jnp.int32),
      pltpu.VMEM((_CH, h), jnp.float32),
      pltpu.VMEM((_CH, h), jnp.float32),
      pltpu.VMEM((_CH, h), jnp.float32),
      pltpu.VMEM_SHARED((np_, h), jnp.float32),
      pltpu.SemaphoreType.DMA,
      pltpu.SemaphoreType.DMA,
      pltpu.SemaphoreType.DMA,
      pltpu.SemaphoreType.DMA,
      pltpu.SemaphoreType.DMA,
      pltpu.SemaphoreType.DMA,
  ]

  def body(h_hbm, pk_hbm, out_hbm, pk_v, srci_v, dsti_v, r0, r1, r2,
           agg_sh, g0, g1, g2, s0, s1, s2):
    c = lax.axis_index("c")
    s = lax.axis_index("s")
    wid = s * _NC + c
    bufs = (r0, r1, r2)
    gsem = (g0, g1, g2)
    ssem = (s0, s1, s2)

    pltpu.sync_copy(pk_hbm.at[wid], pk_v)

    def unpack_src(j, slot):
      for k in range(_CH // 16):
        v = pk_v[j, pl.ds(k * 16, 16)]
        srci_v[slot, pl.ds(k * 16, 16)] = v & 0xFFFF

    def unpack_dst(j, slot):
      for k in range(_CH // 16):
        v = pk_v[j, pl.ds(k * 16, 16)]
        dsti_v[slot, pl.ds(k * 16, 16)] = lax.shift_right_logical(v, 16)

    def issue_gather(j, slot):
      unpack_src(j, slot)
      pltpu.async_copy(h_hbm.at[srci_v.at[slot]], bufs[slot], gsem[slot])

    def wait_gather(slot):
      pltpu.make_async_copy(
          h_hbm.at[srci_v.at[slot]], bufs[slot], gsem[slot]).wait()

    def issue_scatter(j, slot):
      unpack_dst(j, slot)
      pltpu.async_copy(bufs[slot], agg_sh.at[dsti_v.at[slot]], ssem[slot],
                       add=True)

    def wait_scatter(slot):
      pltpu.make_async_copy(
          bufs[slot], agg_sh.at[dsti_v.at[slot]], ssem[slot]).wait()

    def zrow(i, carry):
      for k in range(h // 16):
        r0[i, pl.ds(k * 16, 16)] = jnp.zeros((16,), jnp.float32)
      return carry

    lax.fori_loop(0, _CH, zrow, 0)
    for k in range(nstripe):
      pltpu.sync_copy(r0, agg_sh.at[pl.ds(s * rps + k * _CH, _CH)])
    plsc.subcore_barrier()

    issue_gather(0, 0)

    def step(j, b):
      @pl.when(j < nch)
      def _():
        @pl.when(j >= 2)
        def _():
          wait_scatter((b + 1) % nb)

        @pl.when(j + 1 < nch)
        def _():
          issue_gather(j + 1, (b + 1) % nb)

        wait_gather(b)
        issue_scatter(j, b)

    def group(g, carry):
      for b in range(nb):
        step(g * nb + b, b)
      return carry

    lax.fori_loop(0, (nch + nb - 1) // nb, group, 0)
    for j in (nch - 2, nch - 1):
      wait_scatter(j % nb)
    plsc.subcore_barrier()
    hw = _CH // 2
    for k in range(2 * nstripe):
      rs = s * rps + k * hw
      pltpu.sync_copy(agg_sh.at[pl.ds(rs, hw)], r0.at[pl.ds(0, hw)])
      pltpu.sync_copy(r0.at[pl.ds(0, hw)], out_hbm.at[c].at[pl.ds(rs, hw)])

  return pl.kernel(
      body,
      out_type=jax.ShapeDtypeStruct((_NC, np_, h), jnp.float32),
      mesh=_sc_mesh(), scratch_types=scratch)


def _make_sc_deg(np_, nch):
  rps = np_ // _NS
  nstripe = rps // _CH

  scratch = [
      pltpu.VMEM((nch, _CH), jnp.int32),
      pltpu.VMEM((_CH, 128), jnp.float32),
      pltpu.VMEM_SHARED((np_, 128), jnp.float32),
  ]

  def body(dst_hbm, cnt_hbm, dst_v, ones_v, cnt_sh):
    c = lax.axis_index("c")
    s = lax.axis_index("s")
    wid = s * _NC + c

    pltpu.sync_copy(dst_hbm.at[wid], dst_v)

    def fill(val):
      def fb(i, carry):
        for k in range(8):
          ones_v[i, pl.ds(k * 16, 16)] = jnp.full((16,), val, jnp.float32)
        return carry
      lax.fori_loop(0, _CH, fb, 0)

    fill(0.0)
    for k in range(nstripe):
      pltpu.sync_copy(ones_v, cnt_sh.at[pl.ds(s * rps + k * _CH, _CH)])
    fill(1.0)
    plsc.subcore_barrier()

    def step(j, carry):
      pltpu.sync_copy(ones_v, cnt_sh.at[dst_v.at[j]], add=True)
      return carry

    lax.fori_loop(0, nch, step, 0)
    plsc.subcore_barrier()
    for k in range(nstripe):
      rs = s * rps + k * _CH
      pltpu.sync_copy(cnt_sh.at[pl.ds(rs, _CH)], ones_v)
      pltpu.sync_copy(ones_v, cnt_hbm.at[c].at[pl.ds(rs, _CH)])

  return pl.kernel(
      body,
      out_type=jax.ShapeDtypeStruct((_NC, np_, 128), jnp.float32),
      mesh=_sc_mesh(), scratch_types=scratch)


def _embed_body(x_ref, w_ref, b_ref, o_ref):
  o_ref[...] = (jnp.dot(x_ref[...], w_ref[...],
                        preferred_element_type=jnp.float32) + b_ref[...])


def _combine_norm(p, cnt, h, wl, wr, b, g, be):
  n = h.shape[0]
  agg = p[0][:n] + p[1][:n]
  deg = cnt[0][:n, 0:1] + cnt[1][:n, 0:1]
  inv = 1.0 / jnp.maximum(deg, 1.0)
  y = (jnp.dot(agg * inv, wl, preferred_element_type=jnp.float32)
       + jnp.dot(h, wr, preferred_element_type=jnp.float32) + b)
  mu = jnp.mean(y, axis=0, keepdims=True)
  var = jnp.mean((y - mu) * (y - mu), axis=0, keepdims=True)
  yn = (y - mu) * lax.rsqrt(var + 1e-5) * g + be
  return jnp.maximum(yn, 0.0)


def _layer_body(p_ref, cnt_ref, h_ref, wl_ref, wr_ref, b_ref, g_ref, be_ref,
                o_ref):
  o_ref[...] = _combine_norm(p_ref[...], cnt_ref[...], h_ref[...],
                             wl_ref[...], wr_ref[...], b_ref[...],
                             g_ref[...], be_ref[...])


def _final_body(p_ref, cnt_ref, h_ref, wl_ref, wr_ref, b_ref, g_ref, be_ref,
                wc_ref, bc_ref, o_ref):
  hr = _combine_norm(p_ref[...], cnt_ref[...], h_ref[...],
                     wl_ref[...], wr_ref[...], b_ref[...],
                     g_ref[...], be_ref[...])
  z = jnp.dot(hr, wc_ref[...], preferred_element_type=jnp.float32) + bc_ref[...]
  m = jnp.max(z, axis=-1, keepdims=True)
  ez = jnp.exp(z - m)
  o_ref[...] = ez / jnp.sum(ez, axis=-1, keepdims=True)


def kernel(x, edge_index, W_emb, b_emb, Wl1, Wr1, b1, g1, be1,
           Wl2, Wr2, b2, g2, be2, Wl3, Wr3, b3, g3, be3, Wc, bc):
  n, d = x.shape
  h = W_emb.shape[1]
  o = Wc.shape[1]
  e = edge_index.shape[1]
  epw = e // _NW
  nch = epw // _CH

  blk = _NS * _CH
  np_ = ((n + blk - 1) // blk) * blk

  dst3 = edge_index[1].reshape(_NW, nch, _CH)
  packed3 = jnp.bitwise_or(
      jnp.left_shift(dst3, 16), edge_index[0].reshape(_NW, nch, _CH))

  sc_agg = _make_sc_agg(np_, h, nch)
  sc_deg = _make_sc_deg(np_, nch)

  f32 = jnp.float32
  embed = pl.pallas_call(
      _embed_body, out_shape=jax.ShapeDtypeStruct((n, h), f32))
  layer = pl.pallas_call(
      _layer_body, out_shape=jax.ShapeDtypeStruct((n, h), f32))
  final = pl.pallas_call(
      _final_body, out_shape=jax.ShapeDtypeStruct((n, o), f32))

  cnt = sc_deg(dst3)
  h0 = embed(x, W_emb, b_emb.reshape(1, h))
  p1 = sc_agg(h0, packed3)
  h1 = layer(p1, cnt, h0, Wl1, Wr1, b1.reshape(1, h), g1.reshape(1, h),
             be1.reshape(1, h))
  p2 = sc_agg(h1, packed3)
  h2 = layer(p2, cnt, h1, Wl2, Wr2, b2.reshape(1, h), g2.reshape(1, h),
             be2.reshape(1, h))
  p3 = sc_agg(h2, packed3)
  return final(p3, cnt, h2, Wl3, Wr3, b3.reshape(1, h), g3.reshape(1, h),
               be3.reshape(1, h), Wc, bc.reshape(1, o))


# speedup vs baseline: 11.8022x; 1.0113x over previous
import functools

import jax
import jax.numpy as jnp
from jax import lax
from jax.experimental import pallas as pl
from jax.experimental.pallas import tpu as pltpu
from jax.experimental.pallas import tpu_sc as plsc

_NC, _NS = 2, 16
_NW = _NC * _NS
_CH = 80


_MESH = None


def _sc_mesh():
  global _MESH
  if _MESH is None:
    _MESH = plsc.VectorSubcoreMesh(
        core_axis_name="c", subcore_axis_name="s",
        num_cores=_NC, num_subcores=_NS)
  return _MESH


def _make_sc_agg(np_, h, nch):
  rps = np_ // _NS
  nstripe = rps // _CH

  nb = 3

  scratch = [
      pltpu.VMEM((nch, _CH), jnp.int32),
      pltpu.VMEM((nb, _CH), jnp.int32),
      pltpu.VMEM((nb, _CH), jnp.int32),
      pltpu.VMEM((_CH, h), jnp.float32),
      pltpu.VMEM((_CH, h), jnp.float32),
      pltpu.VMEM((_CH, h), jnp.float32),
      pltpu.VMEM_SHARED((np_, h), jnp.float32),
      pltpu.SemaphoreType.DMA,
      pltpu.SemaphoreType.DMA,
      pltpu.SemaphoreType.DMA,
      pltpu.SemaphoreType.DMA,
      pltpu.SemaphoreType.DMA,
      pltpu.SemaphoreType.DMA,
  ]

  def body(h_hbm, pk_hbm, out_hbm, pk_v, srci_v, dsti_v, r0, r1, r2,
           agg_sh, g0, g1, g2, s0, s1, s2):
    c = lax.axis_index("c")
    s = lax.axis_index("s")
    wid = s * _NC + c
    bufs = (r0, r1, r2)
    gsem = (g0, g1, g2)
    ssem = (s0, s1, s2)

    pltpu.sync_copy(pk_hbm.at[wid], pk_v)

    def unpack_src(j, slot):
      for k in range(_CH // 16):
        v = pk_v[j, pl.ds(k * 16, 16)]
        srci_v[slot, pl.ds(k * 16, 16)] = v & 0xFFFF

    def unpack_dst(j, slot):
      for k in range(_CH // 16):
        v = pk_v[j, pl.ds(k * 16, 16)]
        dsti_v[slot, pl.ds(k * 16, 16)] = lax.shift_right_logical(v, 16)

    def issue_gather(j, slot):
      unpack_src(j, slot)
      pltpu.async_copy(h_hbm.at[srci_v.at[slot]], bufs[slot], gsem[slot])

    def wait_gather(slot):
      pltpu.make_async_copy(
          h_hbm.at[srci_v.at[slot]], bufs[slot], gsem[slot]).wait()

    def issue_scatter(j, slot):
      unpack_dst(j, slot)
      pltpu.async_copy(bufs[slot], agg_sh.at[dsti_v.at[slot]], ssem[slot],
                       add=True)

    def wait_scatter(slot):
      pltpu.make_async_copy(
          bufs[slot], agg_sh.at[dsti_v.at[slot]], ssem[slot]).wait()

    def zrow(i, carry):
      for k in range(h // 16):
        r0[i, pl.ds(k * 16, 16)] = jnp.zeros((16,), jnp.float32)
      return carry

    lax.fori_loop(0, _CH, zrow, 0)
    for k in range(nstripe):
      pltpu.sync_copy(r0, agg_sh.at[pl.ds(s * rps + k * _CH, _CH)])
    plsc.subcore_barrier()

    issue_gather(0, 0)

    def step(j, b):
      @pl.when(j < nch)
      def _():
        @pl.when(j >= 2)
        def _():
          wait_scatter((b + 1) % nb)

        @pl.when(j + 1 < nch)
        def _():
          issue_gather(j + 1, (b + 1) % nb)

        wait_gather(b)
        issue_scatter(j, b)

    def group(g, carry):
      for b in range(nb):
        step(g * nb + b, b)
      return carry

    lax.fori_loop(0, (nch + nb - 1) // nb, group, 0)
    for j in (nch - 2, nch - 1):
      wait_scatter(j % nb)
    plsc.subcore_barrier()
    hw = _CH // 2
    for k in range(2 * nstripe):
      rs = s * rps + k * hw
      pltpu.sync_copy(agg_sh.at[pl.ds(rs, hw)], r0.at[pl.ds(0, hw)])
      pltpu.sync_copy(r0.at[pl.ds(0, hw)], out_hbm.at[c].at[pl.ds(rs, hw)])

  return pl.kernel(
      body,
      out_type=jax.ShapeDtypeStruct((_NC, np_, h), jnp.float32),
      mesh=_sc_mesh(), scratch_types=scratch)


def _make_sc_deg(np_, nch):
  rps = np_ // _NS
  nstripe = rps // _CH
  nb = 3

  scratch = [
      pltpu.VMEM((nch, _CH), jnp.int32),
      pltpu.VMEM((nb, _CH), jnp.int32),
      pltpu.VMEM((_CH, 128), jnp.float32),
      pltpu.VMEM_SHARED((np_, 128), jnp.float32),
      pltpu.SemaphoreType.DMA,
      pltpu.SemaphoreType.DMA,
      pltpu.SemaphoreType.DMA,
  ]

  def body(pk_hbm, cnt_hbm, pk_v, dsti_v, ones_v, cnt_sh, s0, s1, s2):
    c = lax.axis_index("c")
    s = lax.axis_index("s")
    wid = s * _NC + c
    ssem = (s0, s1, s2)

    pltpu.sync_copy(pk_hbm.at[wid], pk_v)

    def fill(val):
      def fb(i, carry):
        for k in range(8):
          ones_v[i, pl.ds(k * 16, 16)] = jnp.full((16,), val, jnp.float32)
        return carry
      lax.fori_loop(0, _CH, fb, 0)

    def wait_scatter(slot):
      pltpu.make_async_copy(
          ones_v, cnt_sh.at[dsti_v.at[slot]], ssem[slot]).wait()

    fill(0.0)
    for k in range(nstripe):
      pltpu.sync_copy(ones_v, cnt_sh.at[pl.ds(s * rps + k * _CH, _CH)])
    fill(1.0)
    plsc.subcore_barrier()

    def step(j, b):
      @pl.when(j < nch)
      def _():
        @pl.when(j >= nb)
        def _():
          wait_scatter(b)
        for k in range(_CH // 16):
          v = pk_v[j, pl.ds(k * 16, 16)]
          dsti_v[b, pl.ds(k * 16, 16)] = lax.shift_right_logical(v, 16)
        pltpu.async_copy(ones_v, cnt_sh.at[dsti_v.at[b]], ssem[b],
                         add=True)

    def group(g, carry):
      for b in range(nb):
        step(g * nb + b, b)
      return carry

    lax.fori_loop(0, (nch + nb - 1) // nb, group, 0)
    for j in (nch - 3, nch - 2, nch - 1):
      wait_scatter(j % nb)
    plsc.subcore_barrier()
    for k in range(nstripe):
      rs = s * rps + k * _CH
      pltpu.sync_copy(cnt_sh.at[pl.ds(rs, _CH)], ones_v)
      pltpu.sync_copy(ones_v, cnt_hbm.at[c].at[pl.ds(rs, _CH)])

  return pl.kernel(
      body,
      out_type=jax.ShapeDtypeStruct((_NC, np_, 128), jnp.float32),
      mesh=_sc_mesh(), scratch_types=scratch)


def _embed_body(x_ref, w_ref, b_ref, o_ref):
  o_ref[...] = (jnp.dot(x_ref[...], w_ref[...],
                        preferred_element_type=jnp.float32) + b_ref[...])


def _combine_norm(p, cnt, h, wl, wr, b, g, be):
  n = h.shape[0]
  agg = p[0][:n] + p[1][:n]
  deg = cnt[0][:n, 0:1] + cnt[1][:n, 0:1]
  inv = 1.0 / jnp.maximum(deg, 1.0)
  y = (jnp.dot(agg * inv, wl, preferred_element_type=jnp.float32)
       + jnp.dot(h, wr, preferred_element_type=jnp.float32) + b)
  mu = jnp.mean(y, axis=0, keepdims=True)
  var = jnp.mean((y - mu) * (y - mu), axis=0, keepdims=True)
  yn = (y - mu) * lax.rsqrt(var + 1e-5) * g + be
  return jnp.maximum(yn, 0.0)


def _layer_body(p_ref, cnt_ref, h_ref, wl_ref, wr_ref, b_ref, g_ref, be_ref,
                o_ref):
  o_ref[...] = _combine_norm(p_ref[...], cnt_ref[...], h_ref[...],
                             wl_ref[...], wr_ref[...], b_ref[...],
                             g_ref[...], be_ref[...])


def _final_body(p_ref, cnt_ref, h_ref, wl_ref, wr_ref, b_ref, g_ref, be_ref,
                wc_ref, bc_ref, o_ref):
  hr = _combine_norm(p_ref[...], cnt_ref[...], h_ref[...],
                     wl_ref[...], wr_ref[...], b_ref[...],
                     g_ref[...], be_ref[...])
  z = jnp.dot(hr, wc_ref[...], preferred_element_type=jnp.float32) + bc_ref[...]
  m = jnp.max(z, axis=-1, keepdims=True)
  ez = jnp.exp(z - m)
  o_ref[...] = ez / jnp.sum(ez, axis=-1, keepdims=True)


def kernel(x, edge_index, W_emb, b_emb, Wl1, Wr1, b1, g1, be1,
           Wl2, Wr2, b2, g2, be2, Wl3, Wr3, b3, g3, be3, Wc, bc):
  n, d = x.shape
  h = W_emb.shape[1]
  o = Wc.shape[1]
  e = edge_index.shape[1]
  epw = e // _NW
  nch = epw // _CH

  blk = _NS * _CH
  np_ = ((n + blk - 1) // blk) * blk

  packed3 = jnp.bitwise_or(
      jnp.left_shift(edge_index[1].reshape(_NW, nch, _CH), 16),
      edge_index[0].reshape(_NW, nch, _CH))

  sc_agg = _make_sc_agg(np_, h, nch)
  sc_deg = _make_sc_deg(np_, nch)

  f32 = jnp.float32
  embed = pl.pallas_call(
      _embed_body, out_shape=jax.ShapeDtypeStruct((n, h), f32))
  layer = pl.pallas_call(
      _layer_body, out_shape=jax.ShapeDtypeStruct((n, h), f32))
  final = pl.pallas_call(
      _final_body, out_shape=jax.ShapeDtypeStruct((n, o), f32))

  cnt = sc_deg(packed3)
  h0 = embed(x, W_emb, b_emb.reshape(1, h))
  p1 = sc_agg(h0, packed3)
  h1 = layer(p1, cnt, h0, Wl1, Wr1, b1.reshape(1, h), g1.reshape(1, h),
             be1.reshape(1, h))
  p2 = sc_agg(h1, packed3)
  h2 = layer(p2, cnt, h1, Wl2, Wr2, b2.reshape(1, h), g2.reshape(1, h),
             be2.reshape(1, h))
  p3 = sc_agg(h2, packed3)
  return final(p3, cnt, h2, Wl3, Wr3, b3.reshape(1, h), g3.reshape(1, h),
               be3.reshape(1, h), Wc, bc.reshape(1, o))
